# baseline (speedup 1.0000x reference)
import functools

import jax
import jax.numpy as jnp
from jax.experimental import pallas as pl
from jax.experimental.pallas import tpu as pltpu

NUM_EXPERTS = 64
TOP_K = 8
AUX_COEF = 0.01
BT = 256
NEG = -1e30


def _router_body(nblk, num_tokens, h_ref, w_ref, out_w_ref, out_i_ref,
                 aux_ref, counts_acc, probs_acc):
    i = pl.program_id(0)

    logits = jax.lax.dot_general(
        h_ref[...], w_ref[...],
        dimension_numbers=(((1,), (1,)), ((), ())),
        preferred_element_type=jnp.float32,
    )

    m = jnp.max(logits, axis=-1, keepdims=True)
    ex = jnp.exp(logits - m)
    probs = ex / jnp.sum(ex, axis=-1, keepdims=True)

    iota = jax.lax.broadcasted_iota(jnp.int32, logits.shape, 1)

    work = logits
    vals, idxs = [], []
    counts = jnp.zeros((1, NUM_EXPERTS), jnp.float32)
    for _ in range(TOP_K):
        vmax = jnp.max(work, axis=-1, keepdims=True)
        sel = jnp.where(work == vmax, iota, NUM_EXPERTS)
        imin = jnp.min(sel, axis=-1, keepdims=True)
        hit = iota == imin
        counts = counts + jnp.sum(hit.astype(jnp.float32), axis=0,
                                  keepdims=True)
        vals.append(vmax)
        idxs.append(imin)
        work = jnp.where(hit, NEG, work)

    top_vals = jnp.concatenate(vals, axis=-1)
    top_idx = jnp.concatenate(idxs, axis=-1)
    exw = jnp.exp(top_vals - top_vals[:, :1])
    out_w_ref[...] = exw / jnp.sum(exw, axis=-1, keepdims=True)
    out_i_ref[...] = top_idx

    prob_part = jnp.sum(probs, axis=0, keepdims=True)

    @pl.when(i == 0)
    def _init():
        counts_acc[...] = counts
        probs_acc[...] = prob_part

    @pl.when(i > 0)
    def _accum():
        counts_acc[...] += counts
        probs_acc[...] += prob_part

    @pl.when(i == nblk - 1)
    def _finalize():
        scale = AUX_COEF * NUM_EXPERTS / (num_tokens * float(num_tokens))
        aux_ref[...] = jnp.sum(counts_acc[...] * probs_acc[...],
                               keepdims=True).reshape(1, 1) * scale


@jax.jit
def kernel(hidden_states, gate_w):
    batch, seq, hidden = hidden_states.shape
    num_tokens = batch * seq
    h_flat = hidden_states.reshape(num_tokens, hidden)
    nblk = num_tokens // BT

    out_w, out_i, aux = pl.pallas_call(
        functools.partial(_router_body, nblk, num_tokens),
        grid=(nblk,),
        in_specs=[
            pl.BlockSpec((BT, hidden), lambda i: (i, 0)),
            pl.BlockSpec((NUM_EXPERTS, hidden), lambda i: (0, 0)),
        ],
        out_specs=[
            pl.BlockSpec((BT, TOP_K), lambda i: (i, 0)),
            pl.BlockSpec((BT, TOP_K), lambda i: (i, 0)),
            pl.BlockSpec((1, 1), lambda i: (0, 0)),
        ],
        out_shape=[
            jax.ShapeDtypeStruct((num_tokens, TOP_K), jnp.float32),
            jax.ShapeDtypeStruct((num_tokens, TOP_K), jnp.int32),
            jax.ShapeDtypeStruct((1, 1), jnp.float32),
        ],
        scratch_shapes=[
            pltpu.VMEM((1, NUM_EXPERTS), jnp.float32),
            pltpu.VMEM((1, NUM_EXPERTS), jnp.float32),
        ],
    )(h_flat, gate_w)

    return (out_w.reshape(batch, seq, TOP_K),
            out_i.reshape(batch, seq, TOP_K),
            aux.reshape(()))


# speedup vs baseline: 1.8256x; 1.8256x over previous
import functools

import jax
import jax.numpy as jnp
from jax.experimental import pallas as pl
from jax.experimental.pallas import tpu as pltpu

NUM_EXPERTS = 64
TOP_K = 8
AUX_COEF = 0.01
BT = 256
NEG = -1e30


def _router_body(nblk, num_tokens, w_ref, h_ref, out_w_ref, out_i_ref,
                 aux_ref, counts_acc, probs_acc):
    i = pl.program_id(0)

    logits = jax.lax.dot_general(
        w_ref[...], h_ref[...],
        dimension_numbers=(((1,), (1,)), ((), ())),
        preferred_element_type=jnp.float32,
    )

    m = jnp.max(logits, axis=0, keepdims=True)
    ex = jnp.exp(logits - m)
    probs = ex / jnp.sum(ex, axis=0, keepdims=True)
    prob_part = jnp.sum(probs, axis=1, keepdims=True)

    iota = jax.lax.broadcasted_iota(jnp.int32, logits.shape, 0)

    work = logits
    vals, idxs = [], []
    for _ in range(TOP_K):
        vmax = jnp.max(work, axis=0, keepdims=True)
        sel = jnp.where(work == vmax, iota, NUM_EXPERTS)
        imin = jnp.min(sel, axis=0, keepdims=True)
        vals.append(vmax)
        idxs.append(imin)
        work = jnp.where(iota == imin, NEG, work)

    top_vals = jnp.concatenate(vals, axis=0)
    top_idx = jnp.concatenate(idxs, axis=0)
    exw = jnp.exp(top_vals - top_vals[0:1, :])
    out_w_ref[...] = exw / jnp.sum(exw, axis=0, keepdims=True)
    out_i_ref[...] = top_idx

    counts = jnp.sum((work == NEG).astype(jnp.float32), axis=1,
                     keepdims=True)

    @pl.when(i == 0)
    def _init():
        counts_acc[...] = counts
        probs_acc[...] = prob_part

    @pl.when(i > 0)
    def _accum():
        counts_acc[...] += counts
        probs_acc[...] += prob_part

    @pl.when(i == nblk - 1)
    def _finalize():
        scale = AUX_COEF * NUM_EXPERTS / (num_tokens * float(num_tokens))
        aux_ref[...] = jnp.sum(counts_acc[...] * probs_acc[...],
                               keepdims=True).reshape(1, 1) * scale


@jax.jit
def kernel(hidden_states, gate_w):
    batch, seq, hidden = hidden_states.shape
    num_tokens = batch * seq
    h_flat = hidden_states.reshape(num_tokens, hidden)
    nblk = num_tokens // BT

    out_w_t, out_i_t, aux = pl.pallas_call(
        functools.partial(_router_body, nblk, num_tokens),
        grid=(nblk,),
        in_specs=[
            pl.BlockSpec((NUM_EXPERTS, hidden), lambda i: (0, 0)),
            pl.BlockSpec((BT, hidden), lambda i: (i, 0)),
        ],
        out_specs=[
            pl.BlockSpec((TOP_K, BT), lambda i: (0, i)),
            pl.BlockSpec((TOP_K, BT), lambda i: (0, i)),
            pl.BlockSpec((1, 1), lambda i: (0, 0)),
        ],
        out_shape=[
            jax.ShapeDtypeStruct((TOP_K, num_tokens), jnp.float32),
            jax.ShapeDtypeStruct((TOP_K, num_tokens), jnp.int32),
            jax.ShapeDtypeStruct((1, 1), jnp.float32),
        ],
        scratch_shapes=[
            pltpu.VMEM((NUM_EXPERTS, 1), jnp.float32),
            pltpu.VMEM((NUM_EXPERTS, 1), jnp.float32),
        ],
    )(gate_w, h_flat)

    return (out_w_t.T.reshape(batch, seq, TOP_K),
            out_i_t.T.reshape(batch, seq, TOP_K),
            aux.reshape(()))


# speedup vs baseline: 2.2910x; 1.2549x over previous
import functools

import jax
import jax.numpy as jnp
from jax.experimental import pallas as pl
from jax.experimental.pallas import tpu as pltpu

NUM_EXPERTS = 64
TOP_K = 8
AUX_COEF = 0.01
BT = 512
NEG = -1e30


def _router_body(nblk, num_tokens, w_ref, h_ref, out_w_ref, out_i_ref,
                 aux_ref, counts_acc, probs_acc):
    i = pl.program_id(0)

    logits = jax.lax.dot_general(
        w_ref[...], h_ref[...],
        dimension_numbers=(((1,), (1,)), ((), ())),
        preferred_element_type=jnp.float32,
    )

    m = jnp.max(logits, axis=0, keepdims=True)
    ex = jnp.exp(logits - m)
    probs = ex / jnp.sum(ex, axis=0, keepdims=True)
    prob_part = jnp.sum(probs, axis=1, keepdims=True)

    iota = jax.lax.broadcasted_iota(jnp.int32, logits.shape, 0)

    work = logits
    vals, idxs = [], []
    for _ in range(TOP_K):
        vmax = jnp.max(work, axis=0, keepdims=True)
        sel = jnp.where(work == vmax, iota, NUM_EXPERTS)
        imin = jnp.min(sel, axis=0, keepdims=True)
        vals.append(vmax)
        idxs.append(imin)
        work = jnp.where(iota == imin, NEG, work)

    top_vals = jnp.concatenate(vals, axis=0)
    top_idx = jnp.concatenate(idxs, axis=0)
    exw = jnp.exp(top_vals - top_vals[0:1, :])
    out_w_ref[...] = exw / jnp.sum(exw, axis=0, keepdims=True)
    out_i_ref[...] = top_idx

    counts = jnp.sum((work == NEG).astype(jnp.float32), axis=1,
                     keepdims=True)

    @pl.when(i == 0)
    def _init():
        counts_acc[...] = counts
        probs_acc[...] = prob_part

    @pl.when(i > 0)
    def _accum():
        counts_acc[...] += counts
        probs_acc[...] += prob_part

    @pl.when(i == nblk - 1)
    def _finalize():
        scale = AUX_COEF * NUM_EXPERTS / (num_tokens * float(num_tokens))
        aux_ref[...] = jnp.sum(counts_acc[...] * probs_acc[...],
                               keepdims=True).reshape(1, 1) * scale


@jax.jit
def kernel(hidden_states, gate_w):
    batch, seq, hidden = hidden_states.shape
    num_tokens = batch * seq
    h_flat = hidden_states.reshape(num_tokens, hidden)
    nblk = num_tokens // BT

    out_w_t, out_i_t, aux = pl.pallas_call(
        functools.partial(_router_body, nblk, num_tokens),
        grid=(nblk,),
        in_specs=[
            pl.BlockSpec((NUM_EXPERTS, hidden), lambda i: (0, 0)),
            pl.BlockSpec((BT, hidden), lambda i: (i, 0)),
        ],
        out_specs=[
            pl.BlockSpec((TOP_K, BT), lambda i: (0, i)),
            pl.BlockSpec((TOP_K, BT), lambda i: (0, i)),
            pl.BlockSpec((1, 1), lambda i: (0, 0)),
        ],
        out_shape=[
            jax.ShapeDtypeStruct((TOP_K, num_tokens), jnp.float32),
            jax.ShapeDtypeStruct((TOP_K, num_tokens), jnp.int32),
            jax.ShapeDtypeStruct((1, 1), jnp.float32),
        ],
        scratch_shapes=[
            pltpu.VMEM((NUM_EXPERTS, 1), jnp.float32),
            pltpu.VMEM((NUM_EXPERTS, 1), jnp.float32),
        ],
    )(gate_w, h_flat)

    return (out_w_t.T.reshape(batch, seq, TOP_K),
            out_i_t.T.reshape(batch, seq, TOP_K),
            aux.reshape(()))


# speedup vs baseline: 2.4435x; 1.0666x over previous
import functools

import jax
import jax.numpy as jnp
from jax.experimental import pallas as pl
from jax.experimental.pallas import tpu as pltpu

NUM_EXPERTS = 64
TOP_K = 8
AUX_COEF = 0.01
BT = 1024
NEG = -1e30


def _router_body(nblk, num_tokens, w_ref, h_ref, out_w_ref, out_i_ref,
                 aux_ref, counts_acc, probs_acc):
    i = pl.program_id(0)

    logits = jax.lax.dot_general(
        w_ref[...], h_ref[...],
        dimension_numbers=(((1,), (1,)), ((), ())),
        preferred_element_type=jnp.float32,
    )

    m = jnp.max(logits, axis=0, keepdims=True)
    ex = jnp.exp(logits - m)
    probs = ex / jnp.sum(ex, axis=0, keepdims=True)
    prob_part = jnp.sum(probs, axis=1, keepdims=True)

    iota = jax.lax.broadcasted_iota(jnp.int32, logits.shape, 0)

    work = logits
    vals, idxs = [], []
    for _ in range(TOP_K):
        vmax = jnp.max(work, axis=0, keepdims=True)
        sel = jnp.where(work == vmax, iota, NUM_EXPERTS)
        imin = jnp.min(sel, axis=0, keepdims=True)
        vals.append(vmax)
        idxs.append(imin)
        work = jnp.where(iota == imin, NEG, work)

    top_vals = jnp.concatenate(vals, axis=0)
    top_idx = jnp.concatenate(idxs, axis=0)
    exw = jnp.exp(top_vals - top_vals[0:1, :])
    out_w_ref[...] = exw / jnp.sum(exw, axis=0, keepdims=True)
    out_i_ref[...] = top_idx

    counts = jnp.sum((work == NEG).astype(jnp.float32), axis=1,
                     keepdims=True)

    @pl.when(i == 0)
    def _init():
        counts_acc[...] = counts
        probs_acc[...] = prob_part

    @pl.when(i > 0)
    def _accum():
        counts_acc[...] += counts
        probs_acc[...] += prob_part

    @pl.when(i == nblk - 1)
    def _finalize():
        scale = AUX_COEF * NUM_EXPERTS / (num_tokens * float(num_tokens))
        aux_ref[...] = jnp.sum(counts_acc[...] * probs_acc[...],
                               keepdims=True).reshape(1, 1) * scale


@jax.jit
def kernel(hidden_states, gate_w):
    batch, seq, hidden = hidden_states.shape
    num_tokens = batch * seq
    h_flat = hidden_states.reshape(num_tokens, hidden)
    nblk = num_tokens // BT

    out_w_t, out_i_t, aux = pl.pallas_call(
        functools.partial(_router_body, nblk, num_tokens),
        grid=(nblk,),
        in_specs=[
            pl.BlockSpec((NUM_EXPERTS, hidden), lambda i: (0, 0)),
            pl.BlockSpec((BT, hidden), lambda i: (i, 0)),
        ],
        out_specs=[
            pl.BlockSpec((TOP_K, BT), lambda i: (0, i)),
            pl.BlockSpec((TOP_K, BT), lambda i: (0, i)),
            pl.BlockSpec((1, 1), lambda i: (0, 0)),
        ],
        out_shape=[
            jax.ShapeDtypeStruct((TOP_K, num_tokens), jnp.float32),
            jax.ShapeDtypeStruct((TOP_K, num_tokens), jnp.int32),
            jax.ShapeDtypeStruct((1, 1), jnp.float32),
        ],
        scratch_shapes=[
            pltpu.VMEM((NUM_EXPERTS, 1), jnp.float32),
            pltpu.VMEM((NUM_EXPERTS, 1), jnp.float32),
        ],
    )(gate_w, h_flat)

    return (out_w_t.T.reshape(batch, seq, TOP_K),
            out_i_t.T.reshape(batch, seq, TOP_K),
            aux.reshape(()))
